# baseline (speedup 1.0000x reference)
import functools

import jax
import jax.numpy as jnp
import numpy as np
from jax.experimental import pallas as pl
from jax.experimental.pallas import tpu as pltpu
from jax.experimental.pallas import tpu_sc as plsc

D = 64
PS = (4, 8, 12, 16, 24, 32, 48, 64)
NE = 8
K = 2
B, T = 128, 2048
GATE_IN = D * (1 + len(PS) * 3)
BB = 2

_INTERPRET = False


def _gelu(v):
  return 0.5 * v * (1.0 + jax.lax.erf(v * np.float32(1.0 / np.sqrt(2.0))))


def _gate_body(x_ref, gW1_ref, gb1_ref, gW2_ref, gb2_ref, wW1_ref, wb1_ref,
               wW2_ref, wb2_ref, acc_ref, idx_ref, coef_ref, wall_ref):
  xb = x_ref[...]
  acc_ref[...] = xb

  p4 = xb.reshape(BB, T // 4, 4, D)
  s = jnp.sum(p4, axis=2)
  q = jnp.sum(p4 * p4, axis=2)
  mn = jnp.min(p4, axis=2)
  mx = jnp.max(p4, axis=2)

  def _pair(a, combine, padv):
    n = a.shape[1]
    if n % 2:
      padb = jnp.full((BB, 1, D), padv, jnp.float32)
      a = jnp.concatenate([a, padb], axis=1)
      n += 1
    a = a.reshape(BB, n // 2, 2, D)
    return combine(a, 2)

  def _triple(a, combine, padv):
    n = a.shape[1]
    r = (-n) % 3
    if r:
      padb = jnp.full((BB, r, D), padv, jnp.float32)
      a = jnp.concatenate([a, padb], axis=1)
      n += r
    a = a.reshape(BB, n // 3, 3, D)
    return combine(a, 2)

  inf = jnp.float32(jnp.inf)
  lv = {4: (s, q, mn, mx)}
  for ps, src, comb in ((8, 4, _pair), (12, 4, _triple), (16, 8, _pair),
                        (24, 12, _pair), (32, 16, _pair), (48, 24, _pair),
                        (64, 32, _pair)):
    ss_, qq_, mn_, mx_ = lv[src]
    lv[ps] = (comb(ss_, jnp.sum, 0.0), comb(qq_, jnp.sum, 0.0),
              comb(mn_, jnp.min, inf), comb(mx_, jnp.max, -inf))

  feats = [jnp.sum(s, axis=1) * np.float32(1.0 / T)]
  for ps in PS:
    ss_, qq_, mn_, mx_ = lv[ps]
    gmin = jnp.min(mn_, axis=1)
    gmax = jnp.max(mx_, axis=1)
    if ps in (12, 24, 48):
      gmin = jnp.minimum(gmin, 0.0)
      gmax = jnp.maximum(gmax, 0.0)
    var = (qq_ - ss_ * ss_ * np.float32(1.0 / ps)) * np.float32(
        1.0 / (ps - 1))
    pstd = jnp.mean(jnp.sqrt(jnp.maximum(var, 0.0)), axis=1)
    feats.extend([gmin, pstd, gmax])
  gate_in = jnp.concatenate(feats, axis=1)

  h = _gelu(
      jnp.dot(gate_in, gW1_ref[...], preferred_element_type=jnp.float32)
      + gb1_ref[0])
  logits = (jnp.dot(h, gW2_ref[...], preferred_element_type=jnp.float32)
            + gb2_ref[0])

  l1 = jnp.max(logits, axis=1)
  i1 = jnp.argmax(logits, axis=1).astype(jnp.int32)
  neg = jnp.float32(-jnp.inf)
  masked = jnp.where(
      jax.lax.broadcasted_iota(jnp.int32, (BB, NE), 1) == i1[:, None],
      neg, logits)
  l2 = jnp.max(masked, axis=1)
  i2 = jnp.argmax(masked, axis=1).astype(jnp.int32)
  e21 = jnp.exp(l2 - l1)
  c1 = 1.0 / (1.0 + e21)
  c2 = 1.0 - c1
  idx_ref[0] = jnp.stack([i1, i2], axis=1)
  coef_ref[0] = jnp.stack([c1, c2], axis=1)

  context = jnp.mean(xb, axis=1)
  ws = []
  for e in range(NE):
    g = _gelu(
        jnp.dot(context, wW1_ref[e], preferred_element_type=jnp.float32)
        + wb1_ref[e])
    v = jnp.dot(g, wW2_ref[e], preferred_element_type=jnp.float32) + wb2_ref[e]
    ws.append(jax.nn.sigmoid(v))
  wall_ref[0] = jnp.concatenate(ws, axis=1)


def _gate_stage(x, gate_params, expert_params):
  gW1, gb1, gW2, gb2 = gate_params
  wW1 = jnp.stack([p[8] for p in expert_params])
  wb1 = jnp.stack([p[9] for p in expert_params])
  wW2 = jnp.stack([p[10] for p in expert_params])
  wb2 = jnp.stack([p[11] for p in expert_params]).reshape(NE, 1)

  grid = (B // BB,)
  acc, idx, coef, wall = pl.pallas_call(
      _gate_body,
      grid=grid,
      in_specs=[
          pl.BlockSpec((BB, T, D), lambda i: (i, 0, 0)),
          pl.BlockSpec((GATE_IN, D), lambda i: (0, 0)),
          pl.BlockSpec((1, D), lambda i: (0, 0)),
          pl.BlockSpec((D, NE), lambda i: (0, 0)),
          pl.BlockSpec((1, NE), lambda i: (0, 0)),
          pl.BlockSpec((NE, D, D), lambda i: (0, 0, 0)),
          pl.BlockSpec((NE, D), lambda i: (0, 0)),
          pl.BlockSpec((NE, D, 1), lambda i: (0, 0, 0)),
          pl.BlockSpec((NE, 1), lambda i: (0, 0)),
      ],
      out_specs=[
          pl.BlockSpec((BB, T, D), lambda i: (i, 0, 0)),
          pl.BlockSpec((1, BB, K), lambda i: (i, 0, 0)),
          pl.BlockSpec((1, BB, K), lambda i: (i, 0, 0)),
          pl.BlockSpec((1, BB, NE), lambda i: (i, 0, 0)),
      ],
      out_shape=[
          jax.ShapeDtypeStruct((B, T, D), jnp.float32),
          jax.ShapeDtypeStruct((B // BB, BB, K), jnp.int32),
          jax.ShapeDtypeStruct((B // BB, BB, K), jnp.float32),
          jax.ShapeDtypeStruct((B // BB, BB, NE), jnp.float32),
      ],
      interpret=_INTERPRET,
  )(x, gW1, gb1.reshape(1, D), gW2, gb2.reshape(1, NE), wW1, wb1, wW2, wb2)
  return acc, idx.reshape(B, K), coef.reshape(B, K), wall.reshape(B, NE)


def _route_body(eid_ref, cf_ref, sid_ref, cnt_ref, cfo_ref):
  a_i = jax.lax.broadcasted_iota(jnp.int32, (1, B * K), 1)
  ev = eid_ref[...]
  m = (ev == jax.lax.broadcasted_iota(
      jnp.int32, (NE, B * K), 0)).astype(jnp.float32)
  lt = (jax.lax.broadcasted_iota(jnp.int32, (B * K, B * K), 0)
        <= jax.lax.broadcasted_iota(
            jnp.int32, (B * K, B * K), 1)).astype(jnp.float32)
  pref = jnp.dot(m, lt, preferred_element_type=jnp.float32)
  slot = pref - 1.0
  cnt_ref[...] = jnp.sum(m, axis=1, keepdims=True).astype(jnp.int32)
  sample = (a_i // K).astype(jnp.float32)
  s_i = jax.lax.broadcasted_iota(jnp.int32, (NE, B * K, B), 2).astype(
      jnp.float32)
  oh = jnp.where(
      jnp.logical_and(slot[:, :, None] == s_i, m[:, :, None] > 0.0),
      1.0, 0.0)
  sid_ref[...] = jnp.sum(
      oh * sample[0][None, :, None], axis=1).astype(jnp.int32)
  cfo_ref[...] = jnp.sum(oh * cf_ref[...][0][None, :, None], axis=1)


def _route(idx, coef):
  eids = idx.reshape(1, B * K).astype(jnp.int32)
  gates = coef.reshape(1, B * K)
  sid, cnt, cf = pl.pallas_call(
      _route_body,
      out_shape=[
          jax.ShapeDtypeStruct((NE, B), jnp.int32),
          jax.ShapeDtypeStruct((NE, 1), jnp.int32),
          jax.ShapeDtypeStruct((NE, B), jnp.float32),
      ],
      interpret=_INTERPRET,
  )(eids, gates)
  return sid, cnt.reshape(NE), cf


def _expert_body(sid_ref, cnt_ref, cf_ref, wall_ref,
                 x_ref, acc_ref, iW1_ref, ib1_ref, iW2_ref, ib2_ref,
                 rW1_ref, rb1_ref, rW2_ref, rb2_ref, out_ref,
                 *, e: int, ps: int):
  pad = (-T) % ps
  tp = T + pad
  n = tp // ps
  i = pl.program_id(0)
  cnt = cnt_ref[0]

  @pl.when(i < cnt)
  def _():
    b = sid_ref[i]
    w = wall_ref[b * NE + e]
    cfv = cf_ref[i]
    xb = x_ref[0]
    if pad:
      xp = jnp.concatenate([xb, jnp.zeros((pad, D), jnp.float32)], axis=0)
    else:
      xp = xb
    xp3 = xp.reshape(n, ps, D)
    h1 = ib1_ref[0]
    for j in range(ps):
      h1 = h1 + jnp.dot(xp3[:, j, :], iW1_ref[j * D:(j + 1) * D, :],
                        preferred_element_type=jnp.float32)
    pm = jnp.mean(xp3, axis=1)
    h2 = (jnp.dot(pm, rW1_ref[...], preferred_element_type=jnp.float32)
          + rb1_ref[0])
    inter = (jnp.dot(h2, rW2_ref[...], preferred_element_type=jnp.float32)
             + rb2_ref[0])
    inter_part = (1.0 - w) * inter
    pieces = []
    for j in range(ps):
      cj = (jnp.dot(h1, iW2_ref[:, j * D:(j + 1) * D],
                    preferred_element_type=jnp.float32)
            + ib2_ref[0, j * D:(j + 1) * D])
      pieces.append((w * cj + inter_part)[:, None, :])
    fused = jnp.concatenate(pieces, axis=1)
    outr = fused.reshape(tp, D)
    if pad:
      outr = outr[:T]
    out_ref[0] = acc_ref[0] + cfv * outr

  @pl.when(jnp.logical_and(i == 0, cnt == 0))
  def _():
    out_ref[0] = acc_ref[0]


def _expert_stage(e, ps, x, acc, sid, cnt, cf, wall_flat, p):
  iW1, ib1, iW2, ib2, rW1, rb1, rW2, rb2 = p[:8]

  def row_map(i, sid_ref, cnt_ref, cf_ref, wall_ref):
    j = jnp.minimum(i, jnp.maximum(cnt_ref[0] - 1, 0))
    return (sid_ref[j], 0, 0)

  def const2(i, *_):
    return (0, 0)

  grid_spec = pltpu.PrefetchScalarGridSpec(
      num_scalar_prefetch=4,
      grid=(B,),
      in_specs=[
          pl.BlockSpec((1, T, D), row_map),
          pl.BlockSpec((1, T, D), row_map),
          pl.BlockSpec((ps * D, D), const2),
          pl.BlockSpec((1, D), const2),
          pl.BlockSpec((D, ps * D), const2),
          pl.BlockSpec((1, ps * D), const2),
          pl.BlockSpec((D, D), const2),
          pl.BlockSpec((1, D), const2),
          pl.BlockSpec((D, D), const2),
          pl.BlockSpec((1, D), const2),
      ],
      out_specs=pl.BlockSpec((1, T, D), row_map),
  )
  out = pl.pallas_call(
      functools.partial(_expert_body, e=e, ps=ps),
      grid_spec=grid_spec,
      out_shape=jax.ShapeDtypeStruct((B, T, D), jnp.float32),
      input_output_aliases={5: 0},
      compiler_params=pltpu.CompilerParams(
          dimension_semantics=("arbitrary",)),
      interpret=_INTERPRET,
  )(sid[e], cnt[e:e + 1], cf[e], wall_flat,
    x, acc, iW1, ib1.reshape(1, D), iW2, ib2.reshape(1, ps * D),
    rW1, rb1.reshape(1, D), rW2, rb2.reshape(1, D))
  return out


def kernel(x, gate_params, expert_params):
  acc, idx, coef, wall = _gate_stage(x, gate_params, expert_params)
  sid, cnt, cf = _route(idx, coef)
  wall_flat = wall.reshape(-1)
  for e, ps in enumerate(PS):
    acc = _expert_stage(e, ps, x, acc, sid, cnt, cf, wall_flat,
                        expert_params[e])
  return acc


# speedup vs baseline: 1.0549x; 1.0549x over previous
import functools

import jax
import jax.numpy as jnp
import numpy as np
from jax.experimental import pallas as pl
from jax.experimental.pallas import tpu as pltpu
from jax.experimental.pallas import tpu_sc as plsc

D = 64
PS = (4, 8, 12, 16, 24, 32, 48, 64)
NE = 8
K = 2
B, T = 128, 2048
GATE_IN = D * (1 + len(PS) * 3)
BB = 2

_INTERPRET = False


def _gelu(v):
  return 0.5 * v * (1.0 + jax.lax.erf(v * np.float32(1.0 / np.sqrt(2.0))))


def _gate_body(x_ref, gW1_ref, gb1_ref, gW2_ref, gb2_ref, wW1_ref, wb1_ref,
               wW2_ref, wb2_ref, acc_ref, idx_ref, coef_ref, wall_ref):
  xb = x_ref[...]
  acc_ref[...] = xb

  p4 = xb.reshape(BB, T // 4, 4, D)
  c0, c1_, c2_, c3 = (p4[:, :, j, :] for j in range(4))
  s = c0 + c1_ + c2_ + c3
  q = c0 * c0 + c1_ * c1_ + c2_ * c2_ + c3 * c3
  mn = jnp.minimum(jnp.minimum(c0, c1_), jnp.minimum(c2_, c3))
  mx = jnp.maximum(jnp.maximum(c0, c1_), jnp.maximum(c2_, c3))

  def _gather_k(a, k, padv):
    n = a.shape[1]
    r = (-n) % k
    if r:
      padb = jnp.full((BB, r, D), padv, jnp.float32)
      a = jnp.concatenate([a, padb], axis=1)
      n += r
    a = a.reshape(BB, n // k, k, D)
    return [a[:, :, j, :] for j in range(k)]

  def _pair(a, combine, padv):
    u = _gather_k(a, 2, padv)
    return combine(u[0], u[1])

  def _triple(a, combine, padv):
    u = _gather_k(a, 3, padv)
    return combine(combine(u[0], u[1]), u[2])

  inf = jnp.float32(jnp.inf)
  lv = {4: (s, q, mn, mx)}
  for ps, src, comb in ((8, 4, _pair), (12, 4, _triple), (16, 8, _pair),
                        (24, 12, _pair), (32, 16, _pair), (48, 24, _pair),
                        (64, 32, _pair)):
    ss_, qq_, mn_, mx_ = lv[src]
    lv[ps] = (comb(ss_, jnp.add, 0.0), comb(qq_, jnp.add, 0.0),
              comb(mn_, jnp.minimum, inf), comb(mx_, jnp.maximum, -inf))

  feats = [jnp.sum(s, axis=1) * np.float32(1.0 / T)]
  for ps in PS:
    ss_, qq_, mn_, mx_ = lv[ps]
    gmin = jnp.min(mn_, axis=1)
    gmax = jnp.max(mx_, axis=1)
    if ps in (12, 24, 48):
      gmin = jnp.minimum(gmin, 0.0)
      gmax = jnp.maximum(gmax, 0.0)
    var = (qq_ - ss_ * ss_ * np.float32(1.0 / ps)) * np.float32(
        1.0 / (ps - 1))
    pstd = jnp.mean(jnp.sqrt(jnp.maximum(var, 0.0)), axis=1)
    feats.extend([gmin, pstd, gmax])
  gate_in = jnp.concatenate(feats, axis=1)

  h = _gelu(
      jnp.dot(gate_in, gW1_ref[...], preferred_element_type=jnp.float32)
      + gb1_ref[0])
  logits = (jnp.dot(h, gW2_ref[...], preferred_element_type=jnp.float32)
            + gb2_ref[0])

  l1 = jnp.max(logits, axis=1)
  i1 = jnp.argmax(logits, axis=1).astype(jnp.int32)
  neg = jnp.float32(-jnp.inf)
  masked = jnp.where(
      jax.lax.broadcasted_iota(jnp.int32, (BB, NE), 1) == i1[:, None],
      neg, logits)
  l2 = jnp.max(masked, axis=1)
  i2 = jnp.argmax(masked, axis=1).astype(jnp.int32)
  e21 = jnp.exp(l2 - l1)
  c1 = 1.0 / (1.0 + e21)
  c2 = 1.0 - c1
  idx_ref[0] = jnp.stack([i1, i2], axis=1)
  coef_ref[0] = jnp.stack([c1, c2], axis=1)

  context = jnp.mean(xb, axis=1)
  ws = []
  for e in range(NE):
    g = _gelu(
        jnp.dot(context, wW1_ref[e], preferred_element_type=jnp.float32)
        + wb1_ref[e])
    v = jnp.dot(g, wW2_ref[e], preferred_element_type=jnp.float32) + wb2_ref[e]
    ws.append(jax.nn.sigmoid(v))
  wall_ref[0] = jnp.concatenate(ws, axis=1)


def _gate_stage(x, gate_params, expert_params):
  gW1, gb1, gW2, gb2 = gate_params
  wW1 = jnp.stack([p[8] for p in expert_params])
  wb1 = jnp.stack([p[9] for p in expert_params])
  wW2 = jnp.stack([p[10] for p in expert_params])
  wb2 = jnp.stack([p[11] for p in expert_params]).reshape(NE, 1)

  grid = (B // BB,)
  acc, idx, coef, wall = pl.pallas_call(
      _gate_body,
      grid=grid,
      in_specs=[
          pl.BlockSpec((BB, T, D), lambda i: (i, 0, 0)),
          pl.BlockSpec((GATE_IN, D), lambda i: (0, 0)),
          pl.BlockSpec((1, D), lambda i: (0, 0)),
          pl.BlockSpec((D, NE), lambda i: (0, 0)),
          pl.BlockSpec((1, NE), lambda i: (0, 0)),
          pl.BlockSpec((NE, D, D), lambda i: (0, 0, 0)),
          pl.BlockSpec((NE, D), lambda i: (0, 0)),
          pl.BlockSpec((NE, D, 1), lambda i: (0, 0, 0)),
          pl.BlockSpec((NE, 1), lambda i: (0, 0)),
      ],
      out_specs=[
          pl.BlockSpec((BB, T, D), lambda i: (i, 0, 0)),
          pl.BlockSpec((1, BB, K), lambda i: (i, 0, 0)),
          pl.BlockSpec((1, BB, K), lambda i: (i, 0, 0)),
          pl.BlockSpec((1, BB, NE), lambda i: (i, 0, 0)),
      ],
      out_shape=[
          jax.ShapeDtypeStruct((B, T, D), jnp.float32),
          jax.ShapeDtypeStruct((B // BB, BB, K), jnp.int32),
          jax.ShapeDtypeStruct((B // BB, BB, K), jnp.float32),
          jax.ShapeDtypeStruct((B // BB, BB, NE), jnp.float32),
      ],
      interpret=_INTERPRET,
  )(x, gW1, gb1.reshape(1, D), gW2, gb2.reshape(1, NE), wW1, wb1, wW2, wb2)
  return acc, idx.reshape(B, K), coef.reshape(B, K), wall.reshape(B, NE)


def _route_body(eid_ref, cf_ref, sid_ref, cnt_ref, cfo_ref):
  a_i = jax.lax.broadcasted_iota(jnp.int32, (1, B * K), 1)
  ev = eid_ref[...]
  m = (ev == jax.lax.broadcasted_iota(
      jnp.int32, (NE, B * K), 0)).astype(jnp.float32)
  lt = (jax.lax.broadcasted_iota(jnp.int32, (B * K, B * K), 0)
        <= jax.lax.broadcasted_iota(
            jnp.int32, (B * K, B * K), 1)).astype(jnp.float32)
  pref = jnp.dot(m, lt, preferred_element_type=jnp.float32)
  slot = pref - 1.0
  cnt_ref[...] = jnp.sum(m, axis=1, keepdims=True).astype(jnp.int32)
  sample = (a_i // K).astype(jnp.float32)
  s_i = jax.lax.broadcasted_iota(jnp.int32, (NE, B * K, B), 2).astype(
      jnp.float32)
  oh = jnp.where(
      jnp.logical_and(slot[:, :, None] == s_i, m[:, :, None] > 0.0),
      1.0, 0.0)
  sid_ref[...] = jnp.sum(
      oh * sample[0][None, :, None], axis=1).astype(jnp.int32)
  cfo_ref[...] = jnp.sum(oh * cf_ref[...][0][None, :, None], axis=1)


def _route(idx, coef):
  eids = idx.reshape(1, B * K).astype(jnp.int32)
  gates = coef.reshape(1, B * K)
  sid, cnt, cf = pl.pallas_call(
      _route_body,
      out_shape=[
          jax.ShapeDtypeStruct((NE, B), jnp.int32),
          jax.ShapeDtypeStruct((NE, 1), jnp.int32),
          jax.ShapeDtypeStruct((NE, B), jnp.float32),
      ],
      interpret=_INTERPRET,
  )(eids, gates)
  return sid, cnt.reshape(NE), cf


def _expert_body(sid_ref, cnt_ref, cf_ref, wall_ref,
                 x_ref, acc_ref, iW1_ref, ib1_ref, iW2_ref, ib2_ref,
                 rW1_ref, rb1_ref, rW2_ref, rb2_ref, out_ref,
                 *, e: int, ps: int):
  pad = (-T) % ps
  tp = T + pad
  n = tp // ps
  i = pl.program_id(0)
  cnt = cnt_ref[0]

  @pl.when(i < cnt)
  def _():
    b = sid_ref[i]
    w = wall_ref[b * NE + e]
    cfv = cf_ref[i]
    xb = x_ref[0]
    if pad:
      xp = jnp.concatenate([xb, jnp.zeros((pad, D), jnp.float32)], axis=0)
    else:
      xp = xb
    xp3 = xp.reshape(n, ps, D)
    h1 = ib1_ref[0]
    for j in range(ps):
      h1 = h1 + jnp.dot(xp3[:, j, :], iW1_ref[j * D:(j + 1) * D, :],
                        preferred_element_type=jnp.float32)
    pm = jnp.mean(xp3, axis=1)
    h2 = (jnp.dot(pm, rW1_ref[...], preferred_element_type=jnp.float32)
          + rb1_ref[0])
    inter = (jnp.dot(h2, rW2_ref[...], preferred_element_type=jnp.float32)
             + rb2_ref[0])
    inter_part = (1.0 - w) * inter
    pieces = []
    for j in range(ps):
      cj = (jnp.dot(h1, iW2_ref[:, j * D:(j + 1) * D],
                    preferred_element_type=jnp.float32)
            + ib2_ref[0, j * D:(j + 1) * D])
      pieces.append((w * cj + inter_part)[:, None, :])
    fused = jnp.concatenate(pieces, axis=1)
    outr = fused.reshape(tp, D)
    if pad:
      outr = outr[:T]
    out_ref[0] = acc_ref[0] + cfv * outr

  @pl.when(jnp.logical_and(i == 0, cnt == 0))
  def _():
    out_ref[0] = acc_ref[0]


def _expert_stage(e, ps, x, acc, sid, cnt, cf, wall_flat, p):
  iW1, ib1, iW2, ib2, rW1, rb1, rW2, rb2 = p[:8]

  def row_map(i, sid_ref, cnt_ref, cf_ref, wall_ref):
    j = jnp.minimum(i, jnp.maximum(cnt_ref[0] - 1, 0))
    return (sid_ref[j], 0, 0)

  def const2(i, *_):
    return (0, 0)

  grid_spec = pltpu.PrefetchScalarGridSpec(
      num_scalar_prefetch=4,
      grid=(B,),
      in_specs=[
          pl.BlockSpec((1, T, D), row_map),
          pl.BlockSpec((1, T, D), row_map),
          pl.BlockSpec((ps * D, D), const2),
          pl.BlockSpec((1, D), const2),
          pl.BlockSpec((D, ps * D), const2),
          pl.BlockSpec((1, ps * D), const2),
          pl.BlockSpec((D, D), const2),
          pl.BlockSpec((1, D), const2),
          pl.BlockSpec((D, D), const2),
          pl.BlockSpec((1, D), const2),
      ],
      out_specs=pl.BlockSpec((1, T, D), row_map),
  )
  out = pl.pallas_call(
      functools.partial(_expert_body, e=e, ps=ps),
      grid_spec=grid_spec,
      out_shape=jax.ShapeDtypeStruct((B, T, D), jnp.float32),
      input_output_aliases={5: 0},
      compiler_params=pltpu.CompilerParams(
          dimension_semantics=("arbitrary",)),
      interpret=_INTERPRET,
  )(sid[e], cnt[e:e + 1], cf[e], wall_flat,
    x, acc, iW1, ib1.reshape(1, D), iW2, ib2.reshape(1, ps * D),
    rW1, rb1.reshape(1, D), rW2, rb2.reshape(1, D))
  return out


def kernel(x, gate_params, expert_params):
  acc, idx, coef, wall = _gate_stage(x, gate_params, expert_params)
  sid, cnt, cf = _route(idx, coef)
  wall_flat = wall.reshape(-1)
  for e, ps in enumerate(PS):
    acc = _expert_stage(e, ps, x, acc, sid, cnt, cf, wall_flat,
                        expert_params[e])
  return acc
